# baseline (speedup 1.0000x reference)
import functools

import jax
import jax.numpy as jnp
from jax import lax
from jax.experimental import pallas as pl
from jax.experimental.pallas import tpu as pltpu
from jax.experimental.pallas import tpu_sc as plsc

B = 4096
K = 20
D = 16
S = K + 1
G = 128 // D

NC = 2
NS = 16
NW = NC * NS
BW = B // NW
NG = BW // 16


def _make_main_sc():
    mesh = plsc.VectorSubcoreMesh(core_axis_name="c", subcore_axis_name="s")

    @functools.partial(
        pl.kernel,
        mesh=mesh,
        out_type=[
            jax.ShapeDtypeStruct((NW, BW), jnp.float32),
            jax.ShapeDtypeStruct((NW, BW), jnp.float32),
        ],
        scratch_types=[
            pltpu.VMEM((BW,), jnp.int32),
            pltpu.VMEM((S, BW), jnp.int32),
            pltpu.VMEM((BW,), jnp.int32),
            pltpu.VMEM((S, BW), jnp.int32),
            pltpu.VMEM((BW, 128), jnp.float32),
            pltpu.VMEM((BW, 128), jnp.float32),
            pltpu.VMEM((BW, 128), jnp.float32),
            pltpu.VMEM((D, BW), jnp.float32),
            pltpu.VMEM((BW,), jnp.float32),
            pltpu.VMEM((BW,), jnp.float32),
            pltpu.SemaphoreType.DMA,
            pltpu.SemaphoreType.DMA,
            pltpu.SemaphoreType.DMA,
        ],
        compiler_params=pltpu.CompilerParams(needs_layout_passes=False),
    )
    def _main(ih8, hu8, ii_hbm, hui_hbm, den_out, sc_out,
              ii_v, hui_v, ii_g, hui_g, gb0, gb1, gie, ieT_v,
              den_v, sc_v, sem_ie, sem0, sem1):
        w = lax.axis_index("s") * NC + lax.axis_index("c")
        pltpu.sync_copy(ii_hbm.at[w], ii_v)
        pltpu.sync_copy(hui_hbm.at[w], hui_v)

        def mk_grp_ii(c, _):
            sl = pl.ds(c * 16, 16)
            ii_g[sl] = lax.shift_right_logical(ii_v[sl], 3)
            return 0
        lax.fori_loop(0, NG, mk_grp_ii, 0)

        def mk_grp_hu(t, _):
            j = t // NG
            c = t % NG
            sl = pl.ds(c * 16, 16)
            hui_g[j, sl] = lax.shift_right_logical(hui_v[j, sl], 3)
            return 0
        lax.fori_loop(0, S * NG, mk_grp_hu, 0)

        bufs = (gb0, gb1)
        sems = (sem0, sem1)
        cp_ie = pltpu.async_copy(ih8.at[ii_g], gie, sem_ie)
        cps = {0: pltpu.async_copy(hu8.at[hui_g.at[0]], bufs[0], sems[0])}
        cp_ie.wait()

        lanes = lax.iota(jnp.int32, 16)

        def ex_ie(c, _):
            sl = pl.ds(c * 16, 16)
            rows = c * 16 + lanes
            off = (ii_v[sl] & 7) * D
            for d in range(D):
                ieT_v[d, sl] = plsc.load_gather(gie, [rows, off + d])
            return 0
        lax.fori_loop(0, NG, ex_ie, 0)

        for j in range(S):
            if j + 1 < S:
                nb = (j + 1) % 2
                cps[j + 1] = pltpu.async_copy(
                    hu8.at[hui_g.at[j + 1]], bufs[nb], sems[nb])
            cps[j].wait()
            buf = bufs[j % 2]

            def score_grp(c, _):
                sl = pl.ds(c * 16, 16)
                rows = c * 16 + lanes
                off = (hui_v[j, sl] & 7) * D
                s = plsc.load_gather(buf, [rows, off]) * ieT_v[0, sl]
                for d in range(1, D):
                    s = s + plsc.load_gather(buf, [rows, off + d]) * ieT_v[d, sl]
                if j == 0:
                    sc_v[sl] = s
                elif j == 1:
                    den_v[sl] = jnp.exp(s)
                else:
                    den_v[sl] = den_v[sl] + jnp.exp(s)
                return 0
            lax.fori_loop(0, NG, score_grp, 0)

        pltpu.sync_copy(den_v, den_out.at[w])
        pltpu.sync_copy(sc_v, sc_out.at[w])

    return _main


def _finish_body(den_ref, sc_ref, out_ref):
    nll = (jnp.sum(jnp.log(den_ref[...])) - jnp.sum(sc_ref[...])) / B
    out_ref[0, 0] = nll


def _to_row_major(table):
    flat = table.T.reshape(1000000 * D)
    cols = [flat[d * 1000000:(d + 1) * 1000000] for d in range(D)]
    return jnp.stack(cols, axis=1).reshape(1000000 // G, G * D)


def kernel(inputs, predict, normal, I_H, H_U):
    ih8 = _to_row_major(I_H)
    hu8 = _to_row_major(H_U)
    ii = inputs.reshape(NW, BW).astype(jnp.int32)
    hui = jnp.concatenate(
        [predict.reshape(B, 1), normal.reshape(B, K)], axis=1
    ).astype(jnp.int32)
    hui = hui.reshape(NW, BW, S).transpose(0, 2, 1)
    den, scp = _make_main_sc()(ih8, hu8, ii, hui)
    out = pl.pallas_call(
        _finish_body,
        out_shape=jax.ShapeDtypeStruct((1, 1), jnp.float32),
        out_specs=pl.BlockSpec(memory_space=pltpu.SMEM),
    )(den, scp)
    return out.reshape(1)


# speedup vs baseline: 5.1579x; 5.1579x over previous
import functools

import jax
import jax.numpy as jnp
from jax import lax
from jax.experimental import pallas as pl
from jax.experimental.pallas import tpu as pltpu
from jax.experimental.pallas import tpu_sc as plsc

B = 4096
K = 20
D = 16
S = K + 1
G = 128 // D

NC = 2
NS = 16
NW = NC * NS
BW = B // NW
NG = BW // 16


def _make_main_sc():
    mesh = plsc.VectorSubcoreMesh(core_axis_name="c", subcore_axis_name="s")

    @functools.partial(
        pl.kernel,
        mesh=mesh,
        out_type=[
            jax.ShapeDtypeStruct((NW, BW), jnp.float32),
            jax.ShapeDtypeStruct((NW, BW), jnp.float32),
        ],
        scratch_types=[
            pltpu.VMEM((BW,), jnp.int32),
            pltpu.VMEM((S, BW), jnp.int32),
            pltpu.VMEM((BW,), jnp.int32),
            pltpu.VMEM((S, BW), jnp.int32),
            pltpu.VMEM((BW, 128), jnp.float32),
            pltpu.VMEM((BW, 128), jnp.float32),
            pltpu.VMEM((BW, 128), jnp.float32),
            pltpu.VMEM((D, BW), jnp.float32),
            pltpu.VMEM((BW,), jnp.float32),
            pltpu.VMEM((BW,), jnp.float32),
            pltpu.SemaphoreType.DMA,
            pltpu.SemaphoreType.DMA,
            pltpu.SemaphoreType.DMA,
        ],
        compiler_params=pltpu.CompilerParams(needs_layout_passes=False),
    )
    def _main(ih8, hu8, ii_hbm, hui_hbm, den_out, sc_out,
              ii_v, hui_v, ii_g, hui_g, gb0, gb1, gie, ieT_v,
              den_v, sc_v, sem_ie, sem0, sem1):
        w = lax.axis_index("s") * NC + lax.axis_index("c")
        pltpu.sync_copy(ii_hbm.at[w], ii_v)
        pltpu.sync_copy(hui_hbm.at[w], hui_v)

        def mk_grp_ii(c, _):
            sl = pl.ds(c * 16, 16)
            ii_g[sl] = lax.shift_right_logical(ii_v[sl], 3)
            return 0
        lax.fori_loop(0, NG, mk_grp_ii, 0)

        def mk_grp_hu(t, _):
            j = t // NG
            c = t % NG
            sl = pl.ds(c * 16, 16)
            hui_g[j, sl] = lax.shift_right_logical(hui_v[j, sl], 3)
            return 0
        lax.fori_loop(0, S * NG, mk_grp_hu, 0)

        bufs = (gb0, gb1)
        sems = (sem0, sem1)
        cp_ie = pltpu.async_copy(ih8.at[ii_g], gie, sem_ie)
        cps = {0: pltpu.async_copy(hu8.at[hui_g.at[0]], bufs[0], sems[0])}
        cp_ie.wait()

        lanes = lax.iota(jnp.int32, 16)

        def ex_ie(c, _):
            sl = pl.ds(c * 16, 16)
            rows = c * 16 + lanes
            off = (ii_v[sl] & 7) * D
            for d in range(D):
                ieT_v[d, sl] = plsc.load_gather(gie, [rows, off + d])
            return 0
        lax.fori_loop(0, NG, ex_ie, 0)

        for j in range(S):
            if j + 1 < S:
                nb = (j + 1) % 2
                cps[j + 1] = pltpu.async_copy(
                    hu8.at[hui_g.at[j + 1]], bufs[nb], sems[nb])
            cps[j].wait()
            buf = bufs[j % 2]

            def score_grp(c, _):
                sl = pl.ds(c * 16, 16)
                rows = c * 16 + lanes
                off = (hui_v[j, sl] & 7) * D
                s = plsc.load_gather(buf, [rows, off]) * ieT_v[0, sl]
                for d in range(1, D):
                    s = s + plsc.load_gather(buf, [rows, off + d]) * ieT_v[d, sl]
                if j == 0:
                    sc_v[sl] = s
                elif j == 1:
                    den_v[sl] = jnp.exp(s)
                else:
                    den_v[sl] = den_v[sl] + jnp.exp(s)
                return 0
            lax.fori_loop(0, NG, score_grp, 0)

        pltpu.sync_copy(den_v, den_out.at[w])
        pltpu.sync_copy(sc_v, sc_out.at[w])

    return _main


def _finish_body(den_ref, sc_ref, out_ref):
    nll = (jnp.sum(jnp.log(den_ref[...])) - jnp.sum(sc_ref[...])) / B
    out_ref[0, 0] = nll


def kernel(inputs, predict, normal, I_H, H_U):
    ih8 = I_H.reshape(1000000 // G, G * D)
    hu8 = H_U.reshape(1000000 // G, G * D)
    ii = inputs.reshape(NW, BW).astype(jnp.int32)
    hui = jnp.concatenate(
        [predict.reshape(B, 1), normal.reshape(B, K)], axis=1
    ).astype(jnp.int32)
    hui = hui.reshape(NW, BW, S).transpose(0, 2, 1)
    den, scp = _make_main_sc()(ih8, hu8, ii, hui)
    out = pl.pallas_call(
        _finish_body,
        out_shape=jax.ShapeDtypeStruct((1, 1), jnp.float32),
        out_specs=pl.BlockSpec(memory_space=pltpu.SMEM),
    )(den, scp)
    return out.reshape(1)
